# Initial kernel scaffold; baseline (speedup 1.0000x reference)
import jax
import jax.numpy as jnp
from jax.experimental import pallas as pl


def kernel(x, gamma, beta):
    raise NotImplementedError("write your pallas kernel here")


# speedup vs baseline: 1.2830x; 1.2830x over previous
import functools

import jax
import jax.numpy as jnp
from jax.experimental import pallas as pl
from jax.experimental.pallas import tpu as pltpu

_EPS_LOG = 1e-6
_EPS_BN = 1e-5


def _k1_body(x_ref, py_ref, part_ref):
    j = pl.program_id(1)
    bB, C, L = x_ref.shape
    xb = x_ref[...]
    y = jnp.log(jnp.abs(xb) + _EPS_LOG)
    s = jnp.sum(jnp.sum(y, axis=2, keepdims=True), axis=0, keepdims=True)
    ss = jnp.sum(jnp.sum(y * y, axis=2, keepdims=True), axis=0, keepdims=True)
    part = jnp.concatenate([s, ss], axis=2)[0]

    @pl.when(j == 0)
    def _init():
        part_ref[...] = part[None]

    @pl.when(j > 0)
    def _acc():
        part_ref[...] = part_ref[...] + part[None]

    ii = jax.lax.broadcasted_iota(jnp.int32, (128, 64), 0)
    jj = jax.lax.broadcasted_iota(jnp.int32, (128, 64), 1)
    P = jnp.where((ii >> 1) == jj, 0.5, 0.0).astype(jnp.float32)
    y2 = y.reshape(bB * C, L)
    nfull = L // 128
    for c in range(0, nfull - 1, 2):
        a = jax.lax.dot(y2[:, c * 128:(c + 1) * 128], P,
                        preferred_element_type=jnp.float32)
        b = jax.lax.dot(y2[:, (c + 1) * 128:(c + 2) * 128], P,
                        preferred_element_type=jnp.float32)
        ab = jnp.concatenate([a, b], axis=1)
        py_ref[:, :, c * 64:(c + 2) * 64] = ab.reshape(bB, C, 128)
    rem_lo = (nfull - (nfull % 2)) * 128
    rem = L - rem_lo
    if rem > 0:
        r = jax.lax.dot(y2[:, rem_lo:], P[:rem, :rem // 2],
                        preferred_element_type=jnp.float32)
        py_ref[:, :, rem_lo // 2:] = r.reshape(bB, C, rem // 2)


def _k2_body(py_ref, part_ref, gamma_ref, beta_ref, out_ref, *, inv_n):
    parts = part_ref[...]
    tot = parts[0] + parts[1]
    mean = tot[:, 0:1] * inv_n
    var = tot[:, 1:2] * inv_n - mean * mean
    rstd = jax.lax.rsqrt(var + _EPS_BN)
    scale = gamma_ref[...] * rstd
    bias = beta_ref[...] - mean * scale
    out_ref[...] = py_ref[...] * scale[None] + bias[None]


@jax.jit
def kernel(x, gamma, beta):
    B, C, L = x.shape
    Lh = L // 2
    bB = 4
    while B % (2 * bB) != 0:
        bB //= 2
    nb = B // (2 * bB)

    py, parts = pl.pallas_call(
        _k1_body,
        grid=(2, nb),
        in_specs=[
            pl.BlockSpec((bB, C, L), lambda p, j: (p * nb + j, 0, 0)),
        ],
        out_specs=[
            pl.BlockSpec((bB, C, Lh), lambda p, j: (p * nb + j, 0, 0)),
            pl.BlockSpec((1, C, 2), lambda p, j: (p, 0, 0)),
        ],
        out_shape=[
            jax.ShapeDtypeStruct((B, C, Lh), jnp.float32),
            jax.ShapeDtypeStruct((2, C, 2), jnp.float32),
        ],
        compiler_params=pltpu.CompilerParams(
            dimension_semantics=("parallel", "arbitrary"),
            vmem_limit_bytes=100 * 1024 * 1024,
        ),
    )(x)

    out = pl.pallas_call(
        functools.partial(_k2_body, inv_n=1.0 / (B * L)),
        grid=(2, nb),
        in_specs=[
            pl.BlockSpec((bB, C, Lh), lambda p, j: (p * nb + j, 0, 0)),
            pl.BlockSpec((2, C, 2), lambda p, j: (0, 0, 0)),
            pl.BlockSpec((C, 1), lambda p, j: (0, 0)),
            pl.BlockSpec((C, 1), lambda p, j: (0, 0)),
        ],
        out_specs=pl.BlockSpec((bB, C, Lh), lambda p, j: (p * nb + j, 0, 0)),
        out_shape=jax.ShapeDtypeStruct((B, C, Lh), jnp.float32),
        compiler_params=pltpu.CompilerParams(
            dimension_semantics=("parallel", "arbitrary"),
            vmem_limit_bytes=100 * 1024 * 1024,
        ),
    )(py, parts, gamma[:, None], beta[:, None])
    return out


# speedup vs baseline: 1.3528x; 1.0544x over previous
import functools

import jax
import jax.numpy as jnp
from jax.experimental import pallas as pl
from jax.experimental.pallas import tpu as pltpu

_EPS_LOG = 1e-6
_EPS_BN = 1e-5


def _pool_pairs(y2, L, bB, Cc, Lp):
    ii = jax.lax.broadcasted_iota(jnp.int32, (128, 64), 0)
    jj = jax.lax.broadcasted_iota(jnp.int32, (128, 64), 1)
    P = jnp.where((ii >> 1) == jj, 0.5, 0.0).astype(jnp.float32)
    cols = []
    nfull = L // 128
    for c in range(0, nfull - 1, 2):
        a = jax.lax.dot(y2[:, c * 128:(c + 1) * 128], P,
                        preferred_element_type=jnp.float32)
        b = jax.lax.dot(y2[:, (c + 1) * 128:(c + 2) * 128], P,
                        preferred_element_type=jnp.float32)
        cols.append(jnp.concatenate([a, b], axis=1))
    rem_lo = (nfull - (nfull % 2)) * 128
    rem = L - rem_lo
    if rem > 0:
        r = jax.lax.dot(y2[:, rem_lo:], P[:rem, :rem // 2],
                        preferred_element_type=jnp.float32)
        cols.append(r)
    got = rem_lo // 2 + (rem // 2)
    if got < Lp:
        cols.append(jnp.zeros((y2.shape[0], Lp - got), jnp.float32))
    pooled = jnp.concatenate(cols, axis=1)
    return pooled.reshape(bB, Cc, Lp)


def _body(x_ref, gamma_ref, beta_ref, out_ref, py_buf, part_buf, *, inv_n, L):
    ph = pl.program_id(1)
    j = pl.program_id(2)
    bB, Cc, Lp = py_buf.shape[1:]

    @pl.when(ph == 0)
    def _pass1():
        xb = x_ref[...]
        y = jnp.log(jnp.abs(xb) + _EPS_LOG)
        s = jnp.sum(jnp.sum(y, axis=2, keepdims=True), axis=0, keepdims=True)
        ss = jnp.sum(jnp.sum(y * y, axis=2, keepdims=True),
                     axis=0, keepdims=True)
        part = jnp.concatenate([s, ss], axis=2)[0]

        @pl.when(j == 0)
        def _init():
            part_buf[...] = part

        @pl.when(j > 0)
        def _acc():
            part_buf[...] = part_buf[...] + part

        pooled = _pool_pairs(y.reshape(bB * Cc, L), L, bB, Cc, Lp)
        py_buf[j] = pooled.astype(jnp.bfloat16)

    @pl.when(ph == 1)
    def _pass2():
        tot = part_buf[...]
        mean = tot[:, 0:1] * inv_n
        var = tot[:, 1:2] * inv_n - mean * mean
        rstd = jax.lax.rsqrt(var + _EPS_BN)
        scale = gamma_ref[...] * rstd
        bias = beta_ref[...] - mean * scale
        py = py_buf[j].astype(jnp.float32)
        out_ref[...] = py[:, :, :L // 2] * scale[None] + bias[None]


@jax.jit
def kernel(x, gamma, beta):
    B, C, L = x.shape
    Lh = L // 2
    Lp = (Lh + 127) // 128 * 128
    Cc = C // 2
    bB = 4
    while B % bB != 0:
        bB //= 2
    nb = B // bB

    out = pl.pallas_call(
        functools.partial(_body, inv_n=1.0 / (B * L), L=L),
        grid=(2, 2, nb),
        in_specs=[
            pl.BlockSpec((bB, Cc, L),
                         lambda p, ph, j: (jnp.where(ph == 0, j, nb - 1), p, 0)),
            pl.BlockSpec((Cc, 1), lambda p, ph, j: (p, 0)),
            pl.BlockSpec((Cc, 1), lambda p, ph, j: (p, 0)),
        ],
        out_specs=pl.BlockSpec((bB, Cc, Lh),
                               lambda p, ph, j: (jnp.where(ph == 0, 0, j), p, 0)),
        out_shape=jax.ShapeDtypeStruct((B, C, Lh), jnp.float32),
        scratch_shapes=[
            pltpu.VMEM((nb, bB, Cc, Lp), jnp.bfloat16),
            pltpu.VMEM((Cc, 2), jnp.float32),
        ],
        compiler_params=pltpu.CompilerParams(
            dimension_semantics=("parallel", "arbitrary", "arbitrary"),
            vmem_limit_bytes=60 * 1024 * 1024,
        ),
    )(x, gamma[:, None], beta[:, None])
    return out


# speedup vs baseline: 1.3650x; 1.0090x over previous
import functools

import jax
import jax.numpy as jnp
from jax.experimental import pallas as pl
from jax.experimental.pallas import tpu as pltpu

_EPS_LOG = 1e-6
_EPS_BN = 1e-5
_CHUNK = 512


def _pool_pairs(y2, w):
    ii = jax.lax.broadcasted_iota(jnp.int32, (128, 64), 0)
    jj = jax.lax.broadcasted_iota(jnp.int32, (128, 64), 1)
    P = jnp.where((ii >> 1) == jj, 0.5, 0.0).astype(jnp.float32)
    cols = []
    nfull = w // 128
    for c in range(0, nfull - 1, 2):
        a = jax.lax.dot(y2[:, c * 128:(c + 1) * 128], P,
                        preferred_element_type=jnp.float32)
        b = jax.lax.dot(y2[:, (c + 1) * 128:(c + 2) * 128], P,
                        preferred_element_type=jnp.float32)
        cols.append(jnp.concatenate([a, b], axis=1))
    rem_lo = (nfull - (nfull % 2)) * 128
    rem = w - rem_lo
    if rem > 0:
        r = jax.lax.dot(y2[:, rem_lo:], P[:rem, :rem // 2],
                        preferred_element_type=jnp.float32)
        cols.append(r)
    return jnp.concatenate(cols, axis=1)


def _k1_body(x_ref, py_ref, part_ref):
    j = pl.program_id(1)
    bB, C, L = x_ref.shape
    s_acc = None
    ss_acc = None
    for o in range(0, L, _CHUNK):
        w = min(_CHUNK, L - o)
        xk = x_ref[:, :, o:o + w]
        y = jnp.log(jnp.abs(xk) + _EPS_LOG)
        cs = jnp.sum(jnp.sum(y, axis=2, keepdims=True), axis=0, keepdims=True)
        cq = jnp.sum(jnp.sum(y * y, axis=2, keepdims=True),
                     axis=0, keepdims=True)
        s_acc = cs if s_acc is None else s_acc + cs
        ss_acc = cq if ss_acc is None else ss_acc + cq
        pooled = _pool_pairs(y.reshape(bB * C, w), w)
        py_ref[:, :, o // 2:(o + w) // 2] = (
            pooled.reshape(bB, C, w // 2).astype(jnp.bfloat16))

    part = jnp.concatenate([s_acc, ss_acc], axis=2)[0]

    @pl.when(j == 0)
    def _init():
        part_ref[...] = part[None]

    @pl.when(j > 0)
    def _acc():
        part_ref[...] = part_ref[...] + part[None]


def _k2_body(py_ref, part_ref, gamma_ref, beta_ref, out_ref, *, inv_n):
    parts = part_ref[...]
    tot = parts[0] + parts[1]
    mean = tot[:, 0:1] * inv_n
    var = tot[:, 1:2] * inv_n - mean * mean
    rstd = jax.lax.rsqrt(var + _EPS_BN)
    scale = gamma_ref[...] * rstd
    bias = beta_ref[...] - mean * scale
    out_ref[...] = (py_ref[...].astype(jnp.float32) * scale[None]
                    + bias[None])


@jax.jit
def kernel(x, gamma, beta):
    B, C, L = x.shape
    Lh = L // 2
    bB = 4
    while B % (2 * bB) != 0:
        bB //= 2
    nb = B // (2 * bB)

    py, parts = pl.pallas_call(
        _k1_body,
        grid=(2, nb),
        in_specs=[
            pl.BlockSpec((bB, C, L), lambda p, j: (p * nb + j, 0, 0)),
        ],
        out_specs=[
            pl.BlockSpec((bB, C, Lh), lambda p, j: (p * nb + j, 0, 0)),
            pl.BlockSpec((1, C, 2), lambda p, j: (p, 0, 0)),
        ],
        out_shape=[
            jax.ShapeDtypeStruct((B, C, Lh), jnp.bfloat16),
            jax.ShapeDtypeStruct((2, C, 2), jnp.float32),
        ],
        compiler_params=pltpu.CompilerParams(
            dimension_semantics=("parallel", "arbitrary"),
            vmem_limit_bytes=100 * 1024 * 1024,
        ),
    )(x)

    out = pl.pallas_call(
        functools.partial(_k2_body, inv_n=1.0 / (B * L)),
        grid=(2, nb),
        in_specs=[
            pl.BlockSpec((bB, C, Lh), lambda p, j: (p * nb + j, 0, 0)),
            pl.BlockSpec((2, C, 2), lambda p, j: (0, 0, 0)),
            pl.BlockSpec((C, 1), lambda p, j: (0, 0)),
            pl.BlockSpec((C, 1), lambda p, j: (0, 0)),
        ],
        out_specs=pl.BlockSpec((bB, C, Lh), lambda p, j: (p * nb + j, 0, 0)),
        out_shape=jax.ShapeDtypeStruct((B, C, Lh), jnp.float32),
        compiler_params=pltpu.CompilerParams(
            dimension_semantics=("parallel", "arbitrary"),
            vmem_limit_bytes=100 * 1024 * 1024,
        ),
    )(py, parts, gamma[:, None], beta[:, None])
    return out


# speedup vs baseline: 1.3929x; 1.0204x over previous
import functools

import jax
import jax.numpy as jnp
from jax.experimental import pallas as pl
from jax.experimental.pallas import tpu as pltpu

_EPS_LOG = 1e-6
_EPS_BN = 1e-5
_CHUNK = 512


def _pool_pairs(y2, w):
    ii = jax.lax.broadcasted_iota(jnp.int32, (256, 128), 0)
    jj = jax.lax.broadcasted_iota(jnp.int32, (256, 128), 1)
    P = jnp.where((ii >> 1) == jj, 0.5, 0.0).astype(jnp.bfloat16)
    cols = []
    for o in range(0, w, 256):
        kw = min(256, w - o)
        cols.append(jax.lax.dot(y2[:, o:o + kw], P[:kw, :kw // 2],
                                preferred_element_type=jnp.float32))
    return jnp.concatenate(cols, axis=1).astype(jnp.bfloat16)


def _k1_body(x_ref, py_ref, part_ref):
    j = pl.program_id(1)
    bB, C, L = x_ref.shape
    s_acc = None
    ss_acc = None
    for o in range(0, L, _CHUNK):
        w = min(_CHUNK, L - o)
        xk = x_ref[:, :, o:o + w]
        y = jnp.log(jnp.abs(xk) + _EPS_LOG)
        cs = jnp.sum(jnp.sum(y, axis=2, keepdims=True), axis=0, keepdims=True)
        cq = jnp.sum(jnp.sum(y * y, axis=2, keepdims=True),
                     axis=0, keepdims=True)
        s_acc = cs if s_acc is None else s_acc + cs
        ss_acc = cq if ss_acc is None else ss_acc + cq
        ybf = y.astype(jnp.bfloat16)
        pooled = _pool_pairs(ybf.reshape(bB * C, w), w)
        py_ref[:, :, o // 2:(o + w) // 2] = pooled.reshape(bB, C, w // 2)

    part = jnp.concatenate([s_acc, ss_acc], axis=2)[0]

    @pl.when(j == 0)
    def _init():
        part_ref[...] = part[None]

    @pl.when(j > 0)
    def _acc():
        part_ref[...] = part_ref[...] + part[None]


def _k2_body(py_ref, part_ref, gamma_ref, beta_ref, out_ref, *, inv_n):
    parts = part_ref[...]
    tot = parts[0] + parts[1]
    mean = tot[:, 0:1] * inv_n
    var = tot[:, 1:2] * inv_n - mean * mean
    rstd = jax.lax.rsqrt(var + _EPS_BN)
    scale = gamma_ref[...] * rstd
    bias = beta_ref[...] - mean * scale
    out_ref[...] = (py_ref[...].astype(jnp.float32) * scale[None]
                    + bias[None])


@jax.jit
def kernel(x, gamma, beta):
    B, C, L = x.shape
    Lh = L // 2
    bB = 4
    while B % (2 * bB) != 0:
        bB //= 2
    nb = B // (2 * bB)

    py, parts = pl.pallas_call(
        _k1_body,
        grid=(2, nb),
        in_specs=[
            pl.BlockSpec((bB, C, L), lambda p, j: (p * nb + j, 0, 0)),
        ],
        out_specs=[
            pl.BlockSpec((bB, C, Lh), lambda p, j: (p * nb + j, 0, 0)),
            pl.BlockSpec((1, C, 2), lambda p, j: (p, 0, 0)),
        ],
        out_shape=[
            jax.ShapeDtypeStruct((B, C, Lh), jnp.bfloat16),
            jax.ShapeDtypeStruct((2, C, 2), jnp.float32),
        ],
        compiler_params=pltpu.CompilerParams(
            dimension_semantics=("parallel", "arbitrary"),
            vmem_limit_bytes=100 * 1024 * 1024,
        ),
    )(x)

    out = pl.pallas_call(
        functools.partial(_k2_body, inv_n=1.0 / (B * L)),
        grid=(2, nb),
        in_specs=[
            pl.BlockSpec((bB, C, Lh), lambda p, j: (p * nb + j, 0, 0)),
            pl.BlockSpec((2, C, 2), lambda p, j: (0, 0, 0)),
            pl.BlockSpec((C, 1), lambda p, j: (0, 0)),
            pl.BlockSpec((C, 1), lambda p, j: (0, 0)),
        ],
        out_specs=pl.BlockSpec((bB, C, Lh), lambda p, j: (p * nb + j, 0, 0)),
        out_shape=jax.ShapeDtypeStruct((B, C, Lh), jnp.float32),
        compiler_params=pltpu.CompilerParams(
            dimension_semantics=("parallel", "arbitrary"),
            vmem_limit_bytes=100 * 1024 * 1024,
        ),
    )(py, parts, gamma[:, None], beta[:, None])
    return out
